# Initial kernel scaffold; baseline (speedup 1.0000x reference)
import jax
import jax.numpy as jnp
from jax.experimental import pallas as pl


def kernel(adj_rows, adj_cols, adj_values, embedding, user_embedding, user, gamma_k):
    raise NotImplementedError("write your pallas kernel here")


# speedup vs baseline: 3.2937x; 3.2937x over previous
import functools

import jax
import jax.numpy as jnp
from jax import lax
from jax.experimental import pallas as pl
from jax.experimental.pallas import tpu as pltpu
from jax.experimental.pallas import tpu_sc as plsc

NC = 2
NS = 16
NW = NC * NS
L = 16

EMB = 64
PE = 2 * EMB
G_STEPS = 8
NR = NW * G_STEPS
CK = 128
DB = 56


_GATHER_DN = lax.GatherDimensionNumbers(
    offset_dims=(), collapsed_slice_dims=(0,), start_index_map=(0,))


def _take16(vec, j):
    idx = jnp.full((L, 1), j, dtype=jnp.int32)
    return lax.gather(vec, idx, _GATHER_DN, slice_sizes=(1,),
                      mode=lax.GatherScatterMode.PROMISE_IN_BOUNDS)


def _make_spmm(n_pad, rt, with_base):
    mesh = plsc.VectorSubcoreMesh(core_axis_name="c", subcore_axis_name="s")

    scratch = [
        pltpu.VMEM((rt, PE), jnp.float32),
        pltpu.VMEM((CK,), jnp.int32),
        pltpu.VMEM((CK,), jnp.int32),
        pltpu.VMEM((CK,), jnp.float32),
        pltpu.VMEM((CK, PE), jnp.float32),
        pltpu.VMEM((NR + L,), jnp.int32),
        pltpu.VMEM((NR + L,), jnp.int32),
        pltpu.VMEM((L,), jnp.float32),
        pltpu.VMEM((DB, PE), jnp.float32),
        pltpu.VMEM((DB, PE), jnp.float32),
        pltpu.VMEM((L,), jnp.float32),
        pltpu.SemaphoreType.DMA,
    ]

    @functools.partial(
        pl.kernel,
        out_type=jax.ShapeDtypeStruct((n_pad, PE), jnp.float32),
        mesh=mesh,
        scratch_types=scratch,
    )
    def spmm(rows_h, cols_h, vals_h, x_h, astart_h, ntrip_h, gamma_h,
             g0_h, e_h, t1_h, out_h,
             acc, colb, rowb, valb, xb, b0, b1, gv, eb, t1b, g0v, sem):
        wid = lax.axis_index("s") * NC + lax.axis_index("c")
        zerof = jnp.zeros((L,), jnp.float32)

        pltpu.sync_copy(gamma_h, gv)
        pltpu.sync_copy(g0_h, g0v)
        pltpu.sync_copy(astart_h, b0)
        pltpu.sync_copy(ntrip_h, b1)
        gvec = gv[...]
        g0vec = g0v[...]

        def step_body(t, carry):
            r = t * NW + wid
            lo = pl.multiple_of(r * rt, 8)
            hi = lo + rt

            def zero_row(i, c):
                for q in range(PE // L):
                    acc[i, pl.ds(q * L, L)] = zerof
                return c
            lax.fori_loop(0, rt, zero_row, 0)

            my_start = b0[pl.ds(r, L)][0]
            my_trips = b1[pl.ds(r, L)][0]

            def chunk_body(c, carry2):
                base = pl.multiple_of(my_start + c * CK, 8)
                pltpu.sync_copy(cols_h.at[pl.ds(base, CK)], colb)
                pltpu.sync_copy(rows_h.at[pl.ds(base, CK)], rowb)
                pltpu.sync_copy(vals_h.at[pl.ds(base, CK)], valb)
                pltpu.async_copy(x_h.at[colb], xb, sem).wait()

                def group_body(g, carry3):
                    gb = g * L
                    row16 = rowb[pl.ds(gb, L)]
                    val16 = valb[pl.ds(gb, L)]
                    inrange = (row16 >= lo) & (row16 < hi)
                    val16 = jnp.where(inrange, val16, 0.0)
                    local16 = row16 - lo
                    for j in range(L):
                        vb = _take16(val16, j)
                        lr = local16[j]
                        lr = jnp.minimum(jnp.maximum(lr, 0), rt - 1)
                        for q in range(EMB // L):
                            sl = pl.ds(q * L, L)
                            x = xb[gb + j, sl]
                            plsc.addupdate(acc.at[lr, sl], x * vb)
                    return carry3
                lax.fori_loop(0, CK // L, group_body, 0)
                return carry2
            lax.fori_loop(0, my_trips, chunk_body, 0)

            def drain_body(d, carry4):
                db = d * DB
                if with_base:
                    doff = pl.multiple_of(lo + db, 8)
                    pltpu.sync_copy(e_h.at[pl.ds(doff, DB)], eb)
                    pltpu.sync_copy(t1_h.at[pl.ds(doff, DB)], t1b)
                def drain_row(i, c5):
                    for q in range(EMB // L):
                        sl = pl.ds(q * L, L)
                        v = acc[db + i, sl] * gvec
                        if with_base:
                            v = v + eb[i, sl] * g0vec + t1b[i, sl]
                        acc[db + i, sl] = v
                    return c5
                lax.fori_loop(0, DB, drain_row, 0)
                return carry4
            lax.fori_loop(0, rt // DB, drain_body, 0)
            pltpu.sync_copy(acc, out_h.at[pl.ds(lo, rt)])
            return carry
        lax.fori_loop(0, G_STEPS, step_body, 0)

    return spmm


def _make_user_gather(n_user, batch):
    mesh = plsc.VectorSubcoreMesh(core_axis_name="c", subcore_axis_name="s")
    bw = batch // NW

    @functools.partial(
        pl.kernel,
        out_type=jax.ShapeDtypeStruct((batch, EMB), jnp.float32),
        mesh=mesh,
        scratch_types=[
            pltpu.VMEM((bw,), jnp.int32),
            pltpu.VMEM((bw, PE), jnp.float32),
            pltpu.VMEM((bw, EMB), jnp.float32),
            pltpu.SemaphoreType.DMA,
        ],
    )
    def ugather(table_h, idx_h, out_h, idxb, rowsb, outb, sem):
        wid = lax.axis_index("s") * NC + lax.axis_index("c")
        base = wid * bw

        pltpu.sync_copy(idx_h.at[pl.ds(base, bw)], idxb)
        pltpu.async_copy(table_h.at[idxb], rowsb, sem).wait()
        def copy_row(i, c):
            for q in range(EMB // L):
                outb[i, pl.ds(q * L, L)] = rowsb[i, pl.ds(q * L, L)]
            return c
        lax.fori_loop(0, bw, copy_row, 0)
        pltpu.sync_copy(outb, out_h.at[pl.ds(base, bw)])

    return ugather


def kernel(adj_rows, adj_cols, adj_values, embedding, user_embedding, user,
           gamma_k):
    n_node, emb = embedding.shape
    assert emb == EMB

    rt = -(-n_node // NR)
    rt = -(-rt // DB) * DB
    n_pad = NR * rt

    rows = adj_rows.astype(jnp.int32)
    cols = adj_cols.astype(jnp.int32)
    vals = adj_values.astype(jnp.float32)
    rows_p = jnp.concatenate([rows, jnp.full((CK,), n_node, jnp.int32)])
    cols_p = jnp.concatenate([cols, jnp.zeros((CK,), jnp.int32)])
    vals_p = jnp.concatenate([vals, jnp.zeros((CK,), jnp.float32)])

    bnd = jnp.searchsorted(rows, jnp.arange(NR + 1, dtype=jnp.int32) * rt
                           ).astype(jnp.int32)
    astart = bnd[:-1] & ~jnp.int32(7)
    aend = -((-bnd[1:]) // 8) * 8
    ntrips = -((-(aend - astart)) // CK)
    astart = jnp.pad(astart, (0, L))
    ntrips = jnp.pad(ntrips, (0, L))

    e2 = jnp.pad(embedding.astype(jnp.float32),
                 ((0, n_pad - n_node), (0, PE - EMB)))
    g0 = jnp.full((L,), gamma_k[0], jnp.float32)
    g1 = jnp.full((L,), gamma_k[1], jnp.float32)
    g2 = jnp.full((L,), gamma_k[2], jnp.float32)

    spmm1 = _make_spmm(n_pad, rt, with_base=False)
    t1 = spmm1(rows_p, cols_p, vals_p, e2, astart, ntrips, g1,
               g0, e2, e2)

    spmm2 = _make_spmm(n_pad, rt, with_base=True)
    final = spmm2(rows_p, cols_p, vals_p, t1, astart, ntrips, g2,
                  g0, e2, t1)

    item_embeddings = final[:n_node, :EMB]

    table2 = jnp.pad(user_embedding.astype(jnp.float32),
                     ((0, 0), (0, PE - EMB)))
    ug = _make_user_gather(table2.shape[0], user.shape[0])
    user_embeddings = ug(table2, user.astype(jnp.int32))
    return item_embeddings, user_embeddings


# speedup vs baseline: 4.0752x; 1.2373x over previous
import functools

import jax
import jax.numpy as jnp
from jax import lax
from jax.experimental import pallas as pl
from jax.experimental.pallas import tpu as pltpu
from jax.experimental.pallas import tpu_sc as plsc

NC = 2
NS = 16
NW = NC * NS
L = 16

EMB = 64
PE = EMB
G_STEPS = 8
NR = NW * G_STEPS
CK = 128
DB = 56


_GATHER_DN = lax.GatherDimensionNumbers(
    offset_dims=(), collapsed_slice_dims=(0,), start_index_map=(0,))


def _take16(vec, j):
    idx = jnp.full((L, 1), j, dtype=jnp.int32)
    return lax.gather(vec, idx, _GATHER_DN, slice_sizes=(1,),
                      mode=lax.GatherScatterMode.PROMISE_IN_BOUNDS)


def _make_spmm(n_pad, rt, nnz_pad, with_base):
    mesh = plsc.VectorSubcoreMesh(core_axis_name="c", subcore_axis_name="s")

    scratch = [
        pltpu.VMEM((rt, PE), jnp.float32),
        pltpu.VMEM((CK,), jnp.int32),
        pltpu.VMEM((CK,), jnp.int32),
        pltpu.VMEM((CK,), jnp.int32),
        pltpu.VMEM((CK,), jnp.int32),
        pltpu.VMEM((CK,), jnp.float32),
        pltpu.VMEM((CK,), jnp.float32),
        pltpu.VMEM((CK, PE), jnp.float32),
        pltpu.VMEM((CK, PE), jnp.float32),
        pltpu.VMEM((NR + L,), jnp.int32),
        pltpu.VMEM((NR + L,), jnp.int32),
        pltpu.VMEM((L,), jnp.float32),
        pltpu.VMEM((DB, PE), jnp.float32),
        pltpu.VMEM((DB, PE), jnp.float32),
        pltpu.VMEM((L,), jnp.float32),
        pltpu.SemaphoreType.DMA,
        pltpu.SemaphoreType.DMA,
    ]

    @functools.partial(
        pl.kernel,
        out_type=jax.ShapeDtypeStruct((n_pad, PE), jnp.float32),
        mesh=mesh,
        scratch_types=scratch,
        compiler_params=pltpu.CompilerParams(use_tc_tiling_on_sc=False),
    )
    def spmm(rows_h, cols_h, vals_h, x_h, astart_h, npair_h, gamma_h,
             g0_h, e_h, t1_h, out_h,
             acc, colb0, colb1, rowb0, rowb1, valb0, valb1, xb0, xb1,
             b0, b1, gv, eb, t1b, g0v, sem0, sem1):
        wid = lax.axis_index("s") * NC + lax.axis_index("c")
        zerof = jnp.zeros((L,), jnp.float32)
        bufs = ((colb0, rowb0, valb0, xb0, sem0),
                (colb1, rowb1, valb1, xb1, sem1))

        pltpu.sync_copy(gamma_h, gv)
        pltpu.sync_copy(g0_h, g0v)
        pltpu.sync_copy(astart_h, b0)
        pltpu.sync_copy(npair_h, b1)
        gvec = gv[...]
        g0vec = g0v[...]

        def step_body(t, carry):
            r = t * NW + wid
            lo = pl.multiple_of(r * rt, 8)
            hi = lo + rt

            def zero_row(i, c):
                for q in range(PE // L):
                    acc[i, pl.ds(q * L, L)] = zerof
                return c
            lax.fori_loop(0, rt, zero_row, 0)

            my_start = b0[pl.ds(r, L)][0]
            my_pairs = b1[pl.ds(r, L)][0]

            def fetch_issue(c, b):
                colb, rowb, valb, xb, sem = bufs[b]
                base = jnp.minimum(my_start + c * CK, nnz_pad - CK)
                base = pl.multiple_of(base, 8)
                pltpu.sync_copy(cols_h.at[pl.ds(base, CK)], colb)
                pltpu.sync_copy(rows_h.at[pl.ds(base, CK)], rowb)
                pltpu.sync_copy(vals_h.at[pl.ds(base, CK)], valb)
                pltpu.async_copy(x_h.at[colb], xb, sem)

            for b in range(2):
                fetch_issue(jnp.int32(b), b)

            def pair_body(p, carry2):
                for b in range(2):
                    c = p * 2 + b
                    colb, rowb, valb, xb, sem = bufs[b]
                    pltpu.make_async_copy(x_h.at[colb], xb, sem).wait()

                    def group_body(g, carry3):
                        gb = g * L
                        row16 = rowb[pl.ds(gb, L)]
                        val16 = valb[pl.ds(gb, L)]
                        inrange = (row16 >= lo) & (row16 < hi)
                        val16 = jnp.where(inrange, val16, 0.0)
                        local16 = row16 - lo
                        local16 = jnp.minimum(
                            jnp.maximum(local16, 0), rt - 1)
                        for j in range(L):
                            vb = _take16(val16, j)
                            lr = local16[j]
                            for q in range(EMB // L):
                                sl = pl.ds(q * L, L)
                                x = xb[gb + j, sl]
                                plsc.addupdate(acc.at[lr, sl], x * vb)
                        return carry3
                    lax.fori_loop(0, CK // L, group_body, 0)
                    fetch_issue(c + 2, b)
                return carry2
            lax.fori_loop(0, my_pairs, pair_body, 0)

            for b in range(2):
                colb, _, _, xb, sem = bufs[b]
                pltpu.make_async_copy(x_h.at[colb], xb, sem).wait()

            def drain_body(d, carry4):
                db = d * DB
                if with_base:
                    doff = pl.multiple_of(lo + db, 8)
                    pltpu.sync_copy(e_h.at[pl.ds(doff, DB)], eb)
                    pltpu.sync_copy(t1_h.at[pl.ds(doff, DB)], t1b)
                def drain_row(i, c5):
                    for q in range(EMB // L):
                        sl = pl.ds(q * L, L)
                        v = acc[db + i, sl] * gvec
                        if with_base:
                            v = v + eb[i, sl] * g0vec + t1b[i, sl]
                        acc[db + i, sl] = v
                    return c5
                lax.fori_loop(0, DB, drain_row, 0)
                return carry4
            lax.fori_loop(0, rt // DB, drain_body, 0)
            pltpu.sync_copy(acc, out_h.at[pl.ds(lo, rt)])
            return carry
        lax.fori_loop(0, G_STEPS, step_body, 0)

    return spmm


def _make_user_gather(n_user, batch):
    mesh = plsc.VectorSubcoreMesh(core_axis_name="c", subcore_axis_name="s")
    bw = batch // NW

    @functools.partial(
        pl.kernel,
        out_type=jax.ShapeDtypeStruct((batch, EMB), jnp.float32),
        mesh=mesh,
        scratch_types=[
            pltpu.VMEM((bw,), jnp.int32),
            pltpu.VMEM((bw, EMB), jnp.float32),
            pltpu.SemaphoreType.DMA,
        ],
        compiler_params=pltpu.CompilerParams(use_tc_tiling_on_sc=False),
    )
    def ugather(table_h, idx_h, out_h, idxb, rowsb, sem):
        wid = lax.axis_index("s") * NC + lax.axis_index("c")
        base = wid * bw

        pltpu.sync_copy(idx_h.at[pl.ds(base, bw)], idxb)
        pltpu.async_copy(table_h.at[idxb], rowsb, sem).wait()
        pltpu.sync_copy(rowsb, out_h.at[pl.ds(base, bw)])

    return ugather


def kernel(adj_rows, adj_cols, adj_values, embedding, user_embedding, user,
           gamma_k):
    n_node, emb = embedding.shape
    assert emb == EMB

    rt = -(-n_node // NR)
    rt = -(-rt // DB) * DB
    n_pad = NR * rt

    rows = adj_rows.astype(jnp.int32)
    cols = adj_cols.astype(jnp.int32)
    vals = adj_values.astype(jnp.float32)
    nnz = rows.shape[0]
    nnz_pad = ((nnz + 3 * CK + 7) // 8) * 8
    pad = nnz_pad - nnz
    rows_p = jnp.concatenate([rows, jnp.full((pad,), n_node, jnp.int32)])
    cols_p = jnp.concatenate([cols, jnp.zeros((pad,), jnp.int32)])
    vals_p = jnp.concatenate([vals, jnp.zeros((pad,), jnp.float32)])

    bnd = jnp.searchsorted(rows, jnp.arange(NR + 1, dtype=jnp.int32) * rt
                           ).astype(jnp.int32)
    astart = bnd[:-1] & ~jnp.int32(7)
    aend = -((-bnd[1:]) // 8) * 8
    ntrips = -((-(aend - astart)) // CK)
    npairs = -((-ntrips) // 2)
    astart = jnp.pad(astart, (0, L))
    npairs = jnp.pad(npairs, (0, L))

    e2 = jnp.pad(embedding.astype(jnp.float32), ((0, n_pad - n_node), (0, 0)))
    g0 = jnp.full((L,), gamma_k[0], jnp.float32)
    g1 = jnp.full((L,), gamma_k[1], jnp.float32)
    g2 = jnp.full((L,), gamma_k[2], jnp.float32)

    spmm1 = _make_spmm(n_pad, rt, nnz_pad, with_base=False)
    t1 = spmm1(rows_p, cols_p, vals_p, e2, astart, npairs, g1,
               g0, e2, e2)

    spmm2 = _make_spmm(n_pad, rt, nnz_pad, with_base=True)
    final = spmm2(rows_p, cols_p, vals_p, t1, astart, npairs, g2,
                  g0, e2, t1)

    item_embeddings = final[:n_node]

    table2 = user_embedding.astype(jnp.float32)
    ug = _make_user_gather(table2.shape[0], user.shape[0])
    user_embeddings = ug(table2, user.astype(jnp.int32))
    return item_embeddings, user_embeddings
